# baseline (speedup 1.0000x reference)
import functools

import jax
import jax.numpy as jnp
from jax import lax
from jax.experimental import pallas as pl
from jax.experimental.pallas import tpu as pltpu
from jax.experimental.pallas import tpu_sc as plsc

_K = 1024
_C = 256
_P = 1024
_B = 8
_BETA = 0.25
_N_TOK = _B * _P
_N_ELEM = _N_TOK * _C


def _outer(x_hbm, cb_ref, xq_hbm, idx_hbm, loss_ref, perp_ref,
           acc_d, acc_sq, acc_cnt):
    acc_d[...] = jnp.zeros_like(acc_d)
    acc_sq[...] = jnp.zeros_like(acc_sq)
    acc_cnt[...] = jnp.zeros_like(acc_cnt)

    cb = cb_ref[...]
    cnorm = jnp.sum(cb * cb, axis=1, keepdims=True)
    k2 = lax.broadcasted_iota(jnp.int32, (_K, 2), 0)
    csel = lax.broadcasted_iota(jnp.int32, (_K, 2), 1)
    kcols = jnp.where(csel == 0, k2 >> 3, k2 & 7).astype(jnp.float32)

    def _tile(x_ref, xq_ref, idx_ref):
        xb = x_ref[0]
        xnorm = jnp.sum(xb * xb, axis=0, keepdims=True)
        scores = lax.dot_general(cb, xb, (((1,), (0,)), ((), ())),
                                 preferred_element_type=jnp.float32)
        dist = (xnorm + cnorm) - 2.0 * scores

        mval = jnp.min(dist, axis=0, keepdims=True)
        iota_k = lax.broadcasted_iota(jnp.int32, (_K, _P), 0)
        idx_row = jnp.min(jnp.where(dist == mval, iota_k, _K),
                          axis=0, keepdims=True)

        onehot_t = (iota_k == idx_row).astype(jnp.float32)

        xq = lax.dot_general(cb, onehot_t, (((0,), (0,)), ((), ())),
                             preferred_element_type=jnp.float32)

        parts = lax.dot_general(onehot_t, kcols, (((0,), (0,)), ((), ())),
                                preferred_element_type=jnp.float32)
        idx_col = (parts[:, 0:1] * 8.0 + parts[:, 1:2]).astype(jnp.int32)

        idx_ref[...] = idx_col
        xq_ref[0] = xb + (xq - xb)

        diff = xb - xq
        ones_row = jnp.full((1, _C), 1.0, jnp.float32)
        acc_d[...] += lax.dot_general(ones_row, diff,
                                      (((1,), (0,)), ((), ())),
                                      preferred_element_type=jnp.float32)
        acc_sq[...] += lax.dot_general(ones_row, diff * diff,
                                       (((1,), (0,)), ((), ())),
                                       preferred_element_type=jnp.float32)
        ones_col = jnp.full((_P, 1), 1.0, jnp.float32)
        acc_cnt[...] += lax.dot_general(onehot_t, ones_col,
                                        (((1,), (0,)), ((), ())),
                                        preferred_element_type=jnp.float32)

    pltpu.emit_pipeline(
        _tile,
        grid=(_B,),
        in_specs=[pl.BlockSpec((1, _C, _P), lambda g: (g, 0, 0))],
        out_specs=[
            pl.BlockSpec((1, _C, _P), lambda g: (g, 0, 0)),
            pl.BlockSpec((_P, 1), lambda g: (g, 0)),
        ],
    )(x_hbm, xq_hbm, idx_hbm)

    inv_n = 1.0 / _N_ELEM
    sum_d = jnp.sum(acc_d[...], keepdims=True)
    sum_sq = jnp.sum(acc_sq[...], keepdims=True)
    loss_ref[...] = _BETA * sum_d * inv_n + sum_sq * inv_n
    e_mean = acc_cnt[...] * (1.0 / _N_TOK)
    ent = jnp.sum(e_mean * jnp.log(e_mean + 1e-10), keepdims=True)
    perp_ref[...] = jnp.exp(-ent)


def _tc_part(x3, codebook):
    out_shapes = (
        jax.ShapeDtypeStruct((_B, _C, _P), jnp.float32),
        jax.ShapeDtypeStruct((_N_TOK, 1), jnp.int32),
        jax.ShapeDtypeStruct((1, 1), jnp.float32),
        jax.ShapeDtypeStruct((1, 1), jnp.float32),
    )
    return pl.pallas_call(
        _outer,
        in_specs=[
            pl.BlockSpec(memory_space=pltpu.HBM),
            pl.BlockSpec(memory_space=pltpu.VMEM),
        ],
        out_specs=(
            pl.BlockSpec(memory_space=pltpu.HBM),
            pl.BlockSpec(memory_space=pltpu.HBM),
            pl.BlockSpec(memory_space=pltpu.VMEM),
            pl.BlockSpec(memory_space=pltpu.VMEM),
        ),
        out_shape=out_shapes,
        scratch_shapes=[
            pltpu.VMEM((1, _P), jnp.float32),
            pltpu.VMEM((1, _P), jnp.float32),
            pltpu.VMEM((_K, 1), jnp.float32),
        ],
        compiler_params=pltpu.CompilerParams(
            vmem_limit_bytes=100 * 1024 * 1024),
    )(x3, codebook)


_NW = 32
_BPW = _N_TOK // _NW
_CHUNK = 64


def _sc_enc_kernel(table_hbm, idx_hbm, out_hbm, idx_v, rows_v, sem):
    wid = lax.axis_index("s") * 2 + lax.axis_index("c")
    base = wid * _BPW
    for j in range(_BPW // _CHUNK):
        off = base + j * _CHUNK
        pltpu.sync_copy(idx_hbm.at[pl.ds(off, _CHUNK)], idx_v)
        pltpu.async_copy(table_hbm.at[idx_v], rows_v, sem).wait()
        pltpu.sync_copy(rows_v, out_hbm.at[pl.ds(off, _CHUNK)])


_sc_enc = pl.kernel(
    _sc_enc_kernel,
    out_type=jax.ShapeDtypeStruct((_N_TOK, _K), jnp.float32),
    mesh=plsc.VectorSubcoreMesh(core_axis_name="c", subcore_axis_name="s"),
    scratch_types=[
        pltpu.VMEM((_CHUNK,), jnp.int32),
        pltpu.VMEM((_CHUNK, _K), jnp.float32),
        pltpu.SemaphoreType.DMA,
    ],
)


@jax.jit
def kernel(x, codebook):
    x3 = x.reshape(_B, _C, _P)
    xq, idx, loss, perp = _tc_part(x3, codebook)
    eye = jnp.eye(_K, dtype=jnp.float32)
    enc = _sc_enc(eye, idx.reshape(_N_TOK))
    xq4 = xq.reshape(_B, _C, 32, 32)
    return (xq4, loss[0, 0], perp[0, 0], enc, idx)


# speedup vs baseline: 1.7524x; 1.7524x over previous
import jax
import jax.numpy as jnp
from jax import lax
from jax.experimental import pallas as pl
from jax.experimental.pallas import tpu as pltpu

_K = 1024
_C = 256
_P = 1024
_B = 8
_BETA = 0.25
_N_TOK = _B * _P
_N_ELEM = _N_TOK * _C


def _outer(x_hbm, cb_ref, xq_hbm, enc_hbm, idx_hbm, loss_ref, perp_ref,
           acc_d, acc_sq, acc_cnt):
    acc_d[...] = jnp.zeros_like(acc_d)
    acc_sq[...] = jnp.zeros_like(acc_sq)
    acc_cnt[...] = jnp.zeros_like(acc_cnt)

    cb = cb_ref[...]
    cnorm = jnp.sum(cb * cb, axis=1, keepdims=True)
    k2 = lax.broadcasted_iota(jnp.int32, (_K, 2), 0)
    csel = lax.broadcasted_iota(jnp.int32, (_K, 2), 1)
    kcols = jnp.where(csel == 0, k2 >> 3, k2 & 7).astype(jnp.float32)

    def _tile(x_ref, xq_ref, enc_ref, idx_ref):
        xb = x_ref[0]
        xnorm = jnp.sum(xb * xb, axis=0, keepdims=True)
        scores = lax.dot_general(cb, xb, (((1,), (0,)), ((), ())),
                                 preferred_element_type=jnp.float32)
        dist = (xnorm + cnorm) - 2.0 * scores

        mval = jnp.min(dist, axis=0, keepdims=True)
        iota_k = lax.broadcasted_iota(jnp.int32, (_K, _P), 0)
        idx_row = jnp.min(jnp.where(dist == mval, iota_k, _K),
                          axis=0, keepdims=True)

        onehot_t = (iota_k == idx_row).astype(jnp.float32)

        xq = lax.dot_general(cb, onehot_t, (((0,), (0,)), ((), ())),
                             preferred_element_type=jnp.float32)

        parts = lax.dot_general(onehot_t, kcols, (((0,), (0,)), ((), ())),
                                preferred_element_type=jnp.float32)
        idx_col = (parts[:, 0:1] * 8.0 + parts[:, 1:2]).astype(jnp.int32)

        iota_lane = lax.broadcasted_iota(jnp.int32, (_P, _K), 1)
        onehot_p = (iota_lane == idx_col).astype(jnp.float32)

        enc_ref[...] = onehot_p
        idx_ref[...] = idx_col
        xq_ref[0] = xb + (xq - xb)

        diff = xb - xq
        ones_row = jnp.full((1, _C), 1.0, jnp.float32)
        acc_d[...] += lax.dot_general(ones_row, diff,
                                      (((1,), (0,)), ((), ())),
                                      preferred_element_type=jnp.float32)
        acc_sq[...] += lax.dot_general(ones_row, diff * diff,
                                       (((1,), (0,)), ((), ())),
                                       preferred_element_type=jnp.float32)
        ones_col = jnp.full((_P, 1), 1.0, jnp.float32)
        acc_cnt[...] += lax.dot_general(onehot_t, ones_col,
                                        (((1,), (0,)), ((), ())),
                                        preferred_element_type=jnp.float32)

    pltpu.emit_pipeline(
        _tile,
        grid=(_B,),
        in_specs=[pl.BlockSpec((1, _C, _P), lambda g: (g, 0, 0))],
        out_specs=[
            pl.BlockSpec((1, _C, _P), lambda g: (g, 0, 0)),
            pl.BlockSpec((_P, _K), lambda g: (g, 0)),
            pl.BlockSpec((_P, 1), lambda g: (g, 0)),
        ],
    )(x_hbm, xq_hbm, enc_hbm, idx_hbm)

    inv_n = 1.0 / _N_ELEM
    sum_d = jnp.sum(acc_d[...], keepdims=True)
    sum_sq = jnp.sum(acc_sq[...], keepdims=True)
    loss_ref[...] = _BETA * sum_d * inv_n + sum_sq * inv_n
    e_mean = acc_cnt[...] * (1.0 / _N_TOK)
    ent = jnp.sum(e_mean * jnp.log(e_mean + 1e-10), keepdims=True)
    perp_ref[...] = jnp.exp(-ent)


@jax.jit
def kernel(x, codebook):
    x3 = x.reshape(_B, _C, _P)
    out_shapes = (
        jax.ShapeDtypeStruct((_B, _C, _P), jnp.float32),
        jax.ShapeDtypeStruct((_N_TOK, _K), jnp.float32),
        jax.ShapeDtypeStruct((_N_TOK, 1), jnp.int32),
        jax.ShapeDtypeStruct((1, 1), jnp.float32),
        jax.ShapeDtypeStruct((1, 1), jnp.float32),
    )
    xq, enc, idx, loss, perp = pl.pallas_call(
        _outer,
        in_specs=[
            pl.BlockSpec(memory_space=pltpu.HBM),
            pl.BlockSpec(memory_space=pltpu.VMEM),
        ],
        out_specs=(
            pl.BlockSpec(memory_space=pltpu.HBM),
            pl.BlockSpec(memory_space=pltpu.HBM),
            pl.BlockSpec(memory_space=pltpu.HBM),
            pl.BlockSpec(memory_space=pltpu.VMEM),
            pl.BlockSpec(memory_space=pltpu.VMEM),
        ),
        out_shape=out_shapes,
        scratch_shapes=[
            pltpu.VMEM((1, _P), jnp.float32),
            pltpu.VMEM((1, _P), jnp.float32),
            pltpu.VMEM((_K, 1), jnp.float32),
        ],
        compiler_params=pltpu.CompilerParams(
            vmem_limit_bytes=100 * 1024 * 1024),
    )(x3, codebook)
    xq4 = xq.reshape(_B, _C, 32, 32)
    return (xq4, loss[0, 0], perp[0, 0], enc, idx)
